# baseline (speedup 1.0000x reference)
import functools

import jax
import jax.numpy as jnp
from jax import lax
from jax.experimental import pallas as pl
from jax.experimental.pallas import tpu as pltpu
from jax.experimental.pallas import tpu_sc as plsc

_NUM_CHANNELS = 1000000
_D = 32
_BATCH = 16384
_HIST = 200
_N = _BATCH * _HIST
_IW = 128
_NROWS = _N // _IW
_NC = 2
_NS = 16
_NW = _NC * _NS
_RPW = _NROWS // _NW
_G = 8
_CHUNKS = _RPW // _G
_BPH = _BATCH // _IW


def _sc_fused(ids2d, table):
    mesh = plsc.VectorSubcoreMesh(
        core_axis_name="c", subcore_axis_name="s",
        num_cores=_NC, num_subcores=_NS)

    @functools.partial(
        pl.kernel,
        out_type=jax.ShapeDtypeStruct((_HIST, _D // 8, _BPH, 8, _IW),
                                      jnp.float32),
        mesh=mesh,
        scratch_types=[
            pltpu.VMEM((_G, _IW), jnp.int32),
            pltpu.VMEM((_G, _IW, _D), jnp.float32),
            pltpu.VMEM((_D // 8, _G, 8, _IW), jnp.float32),
            pltpu.SemaphoreType.DMA,
        ],
        compiler_params=pltpu.CompilerParams(
            use_tc_tiling_on_sc=False, needs_layout_passes=False),
    )
    def k(idx_hbm, table_hbm, out_hbm, idx_v, gbuf, sbuf, sem):
        wid = lax.axis_index("s") * _NC + lax.axis_index("c")
        row0 = wid * _RPW
        lanes = jax.lax.iota(jnp.int32, 16)
        rowvecs = [lanes + bg * 16 for bg in range(8)]

        @pl.loop(0, _CHUNKS)
        def _chunk(c):
            u0 = row0 + c * _G
            h = u0 // _BPH
            bblk0 = u0 % _BPH
            pltpu.sync_copy(idx_hbm.at[pl.ds(u0, _G)], idx_v)

            def _clamp_row(j, _):
                def _clamp16(t, _):
                    v = idx_v[j, pl.ds(t * 16, 16)]
                    v = jnp.minimum(jnp.maximum(v, 0), _NUM_CHANNELS - 1)
                    idx_v[j, pl.ds(t * 16, 16)] = v
                    return 0
                return lax.fori_loop(0, _IW // 16, _clamp16, 0)

            lax.fori_loop(0, _G, _clamp_row, 0)

            copies = [
                pltpu.async_copy(
                    table_hbm.at[idx_v.at[g]],
                    gbuf.at[g],
                    sem,
                )
                for g in range(_G)
            ]
            for cp in copies:
                cp.wait()

            def _trans_g(g, _):
                gv = jnp.full((16,), g, jnp.int32)

                def _trans_d(d, _):
                    dv = jnp.full((16,), d, jnp.int32)
                    for bg in range(8):
                        v = plsc.load_gather(gbuf, [gv, rowvecs[bg], dv])
                        sbuf[d // 8, g, d % 8, pl.ds(bg * 16, 16)] = v
                    return 0

                return lax.fori_loop(0, _D, _trans_d, 0)

            lax.fori_loop(0, _G, _trans_g, 0)
            pltpu.sync_copy(sbuf, out_hbm.at[h, :, pl.ds(bblk0, _G)])

    return k(ids2d, table)


def kernel(channel_ids, table):
    ids2d = channel_ids.T.reshape(_NROWS, _IW)
    out5 = _sc_fused(ids2d, table)
    return out5.transpose(2, 4, 0, 1, 3).reshape(_BATCH, _HIST, _D)


# speedup vs baseline: 1.3369x; 1.3369x over previous
import functools

import jax
import jax.numpy as jnp
from jax import lax
from jax.experimental import pallas as pl
from jax.experimental.pallas import tpu as pltpu
from jax.experimental.pallas import tpu_sc as plsc

_NUM_CHANNELS = 1000000
_D = 32
_BATCH = 16384
_HIST = 200
_N = _BATCH * _HIST
_IW = 128
_NROWS = _N // _IW
_NC = 2
_NS = 16
_NW = _NC * _NS
_RPW = _NROWS // _NW
_G = 16
_CHUNKS = _RPW // _G


def _sc_gather(ids2d, table):
    mesh = plsc.VectorSubcoreMesh(
        core_axis_name="c", subcore_axis_name="s",
        num_cores=_NC, num_subcores=_NS)

    @functools.partial(
        pl.kernel,
        out_type=jax.ShapeDtypeStruct((_NROWS, _IW, _D), jnp.float32),
        mesh=mesh,
        scratch_types=[
            pltpu.VMEM((_G, _IW), jnp.int32),
            pltpu.VMEM((_G, _IW, _D), jnp.float32),
            pltpu.SemaphoreType.DMA,
        ],
        compiler_params=pltpu.CompilerParams(use_tc_tiling_on_sc=False),
    )
    def k(idx_hbm, table_hbm, out_hbm, idx_v, rows_v, sem):
        wid = lax.axis_index("s") * _NC + lax.axis_index("c")
        row0 = wid * _RPW

        @pl.loop(0, _CHUNKS)
        def _chunk(c):
            rbase = row0 + c * _G
            pltpu.sync_copy(idx_hbm.at[pl.ds(rbase, _G)], idx_v)

            def _clamp_row(j, _):
                def _clamp16(t, _):
                    v = idx_v[j, pl.ds(t * 16, 16)]
                    v = jnp.minimum(jnp.maximum(v, 0), _NUM_CHANNELS - 1)
                    idx_v[j, pl.ds(t * 16, 16)] = v
                    return 0
                return lax.fori_loop(0, _IW // 16, _clamp16, 0)

            lax.fori_loop(0, _G, _clamp_row, 0)

            copies = [
                pltpu.async_copy(
                    table_hbm.at[idx_v.at[j]],
                    rows_v.at[j],
                    sem,
                )
                for j in range(_G)
            ]
            for cp in copies:
                cp.wait()
            pltpu.sync_copy(rows_v, out_hbm.at[pl.ds(rbase, _G)])

    return k(ids2d, table)


def _tc_transpose(x):
    _BB = 16384

    def body(x_ref, o_ref):
        eye = jnp.eye(_D, dtype=jnp.float32)
        o_ref[0] = lax.dot_general(
            eye, x_ref[0], (((1,), (1,)), ((), ())),
            preferred_element_type=jnp.float32)

    return pl.pallas_call(
        body,
        grid=(_HIST,),
        in_specs=[pl.BlockSpec((1, _BB, _D), lambda h: (h, 0, 0))],
        out_specs=pl.BlockSpec((1, _D, _BB), lambda h: (h, 0, 0)),
        out_shape=jax.ShapeDtypeStruct((_HIST, _D, _BATCH), jnp.float32),
    )(x)


def kernel(channel_ids, table):
    ids2d = channel_ids.T.reshape(_NROWS, _IW)
    inter = _sc_gather(ids2d, table)
    out_t = _tc_transpose(inter.reshape(_HIST, _BATCH, _D))
    return out_t.transpose(2, 0, 1)
